# Initial kernel scaffold; baseline (speedup 1.0000x reference)
import jax
import jax.numpy as jnp
from jax.experimental import pallas as pl


def kernel(input_indices, lookup, emb_weight):
    raise NotImplementedError("write your pallas kernel here")


# speedup vs baseline: 19.8489x; 19.8489x over previous
import jax
import jax.numpy as jnp
from jax import lax
from jax.experimental import pallas as pl
from jax.experimental.pallas import tpu as pltpu
from jax.experimental.pallas import tpu_sc as plsc

_VOCAB = 10000
_ACTIVE = 100
_DIM = 10
_L = 16
_NW = 32
_N = 16384 * 200
_PER_W = _N // _NW
_CHUNK = 2048
_NCHUNK = _PER_W // _CHUNK
_GROUPS = _CHUNK // _L


def _sc_body(idx_hbm, lookup_hbm, emb_hbm, out_hbm,
             lookup_v, emb_v, idx_v, out_v):
    wid = lax.axis_index("s") * 2 + lax.axis_index("c")
    base = wid * _PER_W

    pltpu.sync_copy(lookup_hbm, lookup_v)
    pltpu.sync_copy(emb_hbm, emb_v)

    def chunk_body(c, carry):
        start = base + c * _CHUNK
        pltpu.sync_copy(idx_hbm.at[pl.ds(start, _CHUNK)], idx_v)

        def group_body(g, carry2):
            off = pl.multiple_of(g * _L, _L)
            ids = idx_v[pl.ds(off, _L)]
            rem = plsc.load_gather(lookup_v, [ids])
            rem = jnp.minimum(jnp.maximum(rem, 0), _ACTIVE - 1)
            rows = off + lax.iota(jnp.int32, _L)
            for j in range(_DIM):
                colj = jnp.full((_L,), j, jnp.int32)
                vals = plsc.load_gather(emb_v, [rem, colj])
                plsc.store_scatter(out_v, [rows, colj], vals)
            return carry2

        lax.fori_loop(0, _GROUPS, group_body, 0, unroll=False)
        pltpu.sync_copy(out_v, out_hbm.at[pl.ds(start, _CHUNK)])
        return carry

    lax.fori_loop(0, _NCHUNK, chunk_body, 0, unroll=False)


def kernel(input_indices, lookup, emb_weight):
    idx_flat = input_indices.reshape(-1)
    mesh = plsc.VectorSubcoreMesh(core_axis_name="c", subcore_axis_name="s")
    f = pl.kernel(
        _sc_body,
        out_type=jax.ShapeDtypeStruct((_N, _DIM), jnp.float32),
        mesh=mesh,
        compiler_params=pltpu.CompilerParams(needs_layout_passes=False,
                                             use_tc_tiling_on_sc=False),
        scratch_types=[
            pltpu.VMEM((_VOCAB,), jnp.int32),
            pltpu.VMEM((_ACTIVE, _DIM), jnp.float32),
            pltpu.VMEM((_CHUNK,), jnp.int32),
            pltpu.VMEM((_CHUNK, _DIM), jnp.float32),
        ],
    )
    out = f(idx_flat, lookup, emb_weight)
    return out.reshape(input_indices.shape + (_DIM,))


# speedup vs baseline: 21.2884x; 1.0725x over previous
import jax
import jax.numpy as jnp
from jax import lax
from jax.experimental import pallas as pl
from jax.experimental.pallas import tpu as pltpu
from jax.experimental.pallas import tpu_sc as plsc

_VOCAB = 10000
_ACTIVE = 100
_DIM = 10
_L = 16
_NW = 32
_N = 16384 * 200
_PER_W = _N // _NW
_CHUNK = 2048
_NCHUNK = _PER_W // _CHUNK
_GROUPS = _CHUNK // _L


def _sc_body(idx_hbm, lookup_hbm, emb_hbm, out_hbm,
             lookup_v, emb_v, idx_v, out_v):
    wid = lax.axis_index("s") * 2 + lax.axis_index("c")
    base = wid * _PER_W

    pltpu.sync_copy(lookup_hbm, lookup_v)
    pltpu.sync_copy(emb_hbm, emb_v)

    def chunk_body(c, carry):
        start = base + c * _CHUNK
        pltpu.sync_copy(idx_hbm.at[pl.ds(start, _CHUNK)], idx_v)

        @plsc.parallel_loop(0, _GROUPS, 1, unroll=4)
        def group_body(g):
            off = pl.multiple_of(g * _L, _L)
            ids = idx_v[pl.ds(off, _L)]
            rem = plsc.load_gather(lookup_v, [ids])
            rem = jnp.minimum(jnp.maximum(rem, 0), _ACTIVE - 1)
            rows = off + lax.iota(jnp.int32, _L)
            for j in range(_DIM):
                colj = jnp.full((_L,), j, jnp.int32)
                vals = plsc.load_gather(emb_v, [rem, colj])
                plsc.store_scatter(out_v, [rows, colj], vals)
        pltpu.sync_copy(out_v, out_hbm.at[pl.ds(start, _CHUNK)])
        return carry

    lax.fori_loop(0, _NCHUNK, chunk_body, 0, unroll=False)


def kernel(input_indices, lookup, emb_weight):
    idx_flat = input_indices.reshape(-1)
    mesh = plsc.VectorSubcoreMesh(core_axis_name="c", subcore_axis_name="s")
    f = pl.kernel(
        _sc_body,
        out_type=jax.ShapeDtypeStruct((_N, _DIM), jnp.float32),
        mesh=mesh,
        compiler_params=pltpu.CompilerParams(needs_layout_passes=False,
                                             use_tc_tiling_on_sc=False),
        scratch_types=[
            pltpu.VMEM((_VOCAB,), jnp.int32),
            pltpu.VMEM((_ACTIVE, _DIM), jnp.float32),
            pltpu.VMEM((_CHUNK,), jnp.int32),
            pltpu.VMEM((_CHUNK, _DIM), jnp.float32),
        ],
    )
    out = f(idx_flat, lookup, emb_weight)
    return out.reshape(input_indices.shape + (_DIM,))
